# Initial kernel scaffold; baseline (speedup 1.0000x reference)
import jax
import jax.numpy as jnp
from jax.experimental import pallas as pl


def kernel(binS, binU, binF, embS, embU, embF):
    raise NotImplementedError("write your pallas kernel here")


# speedup vs baseline: 7.2629x; 7.2629x over previous
import functools

import jax
import jax.numpy as jnp
from jax import lax
from jax.experimental import pallas as pl
from jax.experimental.pallas import tpu as pltpu
from jax.experimental.pallas import tpu_sc as plsc

D_MODEL = 128
N_BINS = 8
N_COMB = N_BINS ** 3
BATCH = 16384
HIST = 50
N_ROWS = BATCH * HIST
IDX_COLS = 128
IDX_ROWS = N_ROWS // IDX_COLS

N_WORKERS = 32
PER_W = N_ROWS // N_WORKERS
CHUNK = 512
K_SUB = CHUNK // 128
N_CHUNKS = PER_W // CHUNK
PER_W_IDX_ROWS = PER_W // IDX_COLS


def _table_body(embS_ref, embU_ref, embF_ref, out_ref):
    c = lax.broadcasted_iota(jnp.int32, (N_COMB, N_BINS), 0)
    j = lax.broadcasted_iota(jnp.int32, (N_COMB, N_BINS), 1)
    ohS = jnp.where((c >> 6) == j, 1.0, 0.0)
    ohU = jnp.where(((c >> 3) & 7) == j, 1.0, 0.0)
    ohF = jnp.where((c & 7) == j, 1.0, 0.0)
    out_ref[...] = (
        jnp.dot(ohS, embS_ref[...], preferred_element_type=jnp.float32)
        + jnp.dot(ohU, embU_ref[...], preferred_element_type=jnp.float32)
        + jnp.dot(ohF, embF_ref[...], preferred_element_type=jnp.float32)
    )


def _build_table(embS, embU, embF):
    return pl.pallas_call(
        _table_body,
        out_shape=jax.ShapeDtypeStruct((N_COMB, D_MODEL), jnp.float32),
    )(embS, embU, embF)


def _cidx_body(s_ref, u_ref, f_ref, o_ref):
    o_ref[...] = s_ref[...] * 64 + u_ref[...] * 8 + f_ref[...]


def _combine_idx(binS, binU, binF):
    s = binS.reshape(IDX_ROWS, IDX_COLS)
    u = binU.reshape(IDX_ROWS, IDX_COLS)
    f = binF.reshape(IDX_ROWS, IDX_COLS)
    grid = 8
    blk = IDX_ROWS // grid
    spec = pl.BlockSpec((blk, IDX_COLS), lambda i: (i, 0))
    return pl.pallas_call(
        _cidx_body,
        grid=(grid,),
        in_specs=[spec, spec, spec],
        out_specs=spec,
        out_shape=jax.ShapeDtypeStruct((IDX_ROWS, IDX_COLS), jnp.int32),
    )(s, u, f)


_mesh = plsc.VectorSubcoreMesh(core_axis_name="c", subcore_axis_name="s")


@functools.partial(
    pl.kernel,
    mesh=_mesh,
    out_type=jax.ShapeDtypeStruct((N_ROWS, D_MODEL), jnp.float32),
    scratch_types=[
        pltpu.VMEM((K_SUB, IDX_COLS), jnp.int32),
        pltpu.VMEM((CHUNK, D_MODEL), jnp.float32),
        pltpu.SemaphoreType.DMA,
    ],
)
def _sc_lookup(table_hbm, cidx_hbm, out_hbm, idx_v, rows_v, sem):
    wid = lax.axis_index("s") * 2 + lax.axis_index("c")
    row_base = wid * PER_W
    idx_row_base = wid * PER_W_IDX_ROWS

    def body(i, carry):
        pltpu.sync_copy(
            cidx_hbm.at[pl.ds(idx_row_base + i * K_SUB, K_SUB)], idx_v)
        copies = []
        for j in range(K_SUB):
            copies.append(pltpu.async_copy(
                table_hbm.at[idx_v.at[j]],
                rows_v.at[pl.ds(j * IDX_COLS, IDX_COLS)],
                sem,
            ))
        for cp in copies:
            cp.wait()
        pltpu.sync_copy(
            rows_v, out_hbm.at[pl.ds(row_base + i * CHUNK, CHUNK)])
        return carry

    lax.fori_loop(0, N_CHUNKS, body, 0)


def kernel(binS, binU, binF, embS, embU, embF):
    table = _build_table(embS, embU, embF)
    cidx = _combine_idx(binS, binU, binF)
    out = _sc_lookup(table, cidx)
    return out.reshape(BATCH, HIST, D_MODEL)


# speedup vs baseline: 7.3030x; 1.0055x over previous
import functools

import jax
import jax.numpy as jnp
from jax import lax
from jax.experimental import pallas as pl
from jax.experimental.pallas import tpu as pltpu
from jax.experimental.pallas import tpu_sc as plsc

D_MODEL = 128
N_BINS = 8
N_COMB = N_BINS ** 3
BATCH = 16384
HIST = 50
N_ROWS = BATCH * HIST
IDX_COLS = 128
IDX_ROWS = N_ROWS // IDX_COLS

N_WORKERS = 32
PER_W = N_ROWS // N_WORKERS
CHUNK = 256
K_SUB = CHUNK // 128
N_CHUNKS = PER_W // CHUNK
N_CHUNK_PAIRS = N_CHUNKS // 2
PER_W_IDX_ROWS = PER_W // IDX_COLS


def _table_body(embS_ref, embU_ref, embF_ref, out_ref):
    c = lax.broadcasted_iota(jnp.int32, (N_COMB, N_BINS), 0)
    j = lax.broadcasted_iota(jnp.int32, (N_COMB, N_BINS), 1)
    ohS = jnp.where((c >> 6) == j, 1.0, 0.0)
    ohU = jnp.where(((c >> 3) & 7) == j, 1.0, 0.0)
    ohF = jnp.where((c & 7) == j, 1.0, 0.0)
    out_ref[...] = (
        jnp.dot(ohS, embS_ref[...], preferred_element_type=jnp.float32)
        + jnp.dot(ohU, embU_ref[...], preferred_element_type=jnp.float32)
        + jnp.dot(ohF, embF_ref[...], preferred_element_type=jnp.float32)
    )


def _build_table(embS, embU, embF):
    return pl.pallas_call(
        _table_body,
        out_shape=jax.ShapeDtypeStruct((N_COMB, D_MODEL), jnp.float32),
    )(embS, embU, embF)


def _cidx_body(s_ref, u_ref, f_ref, o_ref):
    o_ref[...] = s_ref[...] * 64 + u_ref[...] * 8 + f_ref[...]


def _combine_idx(binS, binU, binF):
    s = binS.reshape(IDX_ROWS, IDX_COLS)
    u = binU.reshape(IDX_ROWS, IDX_COLS)
    f = binF.reshape(IDX_ROWS, IDX_COLS)
    grid = 8
    blk = IDX_ROWS // grid
    spec = pl.BlockSpec((blk, IDX_COLS), lambda i: (i, 0))
    return pl.pallas_call(
        _cidx_body,
        grid=(grid,),
        in_specs=[spec, spec, spec],
        out_specs=spec,
        out_shape=jax.ShapeDtypeStruct((IDX_ROWS, IDX_COLS), jnp.int32),
    )(s, u, f)


_mesh = plsc.VectorSubcoreMesh(core_axis_name="c", subcore_axis_name="s")


@functools.partial(
    pl.kernel,
    mesh=_mesh,
    out_type=jax.ShapeDtypeStruct((N_ROWS, D_MODEL), jnp.float32),
    scratch_types=[
        pltpu.VMEM((K_SUB, IDX_COLS), jnp.int32),
        pltpu.VMEM((K_SUB, IDX_COLS), jnp.int32),
        pltpu.VMEM((CHUNK, D_MODEL), jnp.float32),
        pltpu.VMEM((CHUNK, D_MODEL), jnp.float32),
        pltpu.SemaphoreType.DMA,
        pltpu.SemaphoreType.DMA,
        pltpu.SemaphoreType.DMA,
        pltpu.SemaphoreType.DMA,
    ],
)
def _sc_lookup(table_hbm, cidx_hbm, out_hbm,
               idx0, idx1, rows0, rows1, sem_g0, sem_g1, sem_s0, sem_s1):
    wid = lax.axis_index("s") * 2 + lax.axis_index("c")
    row_base = wid * PER_W
    idx_row_base = wid * PER_W_IDX_ROWS

    def load_idx(i, idx_v):
        pltpu.sync_copy(
            cidx_hbm.at[pl.ds(idx_row_base + i * K_SUB, K_SUB)], idx_v)

    def fire_gathers(idx_v, rows_v, sem):
        for j in range(K_SUB):
            pltpu.async_copy(
                table_hbm.at[idx_v.at[j]],
                rows_v.at[pl.ds(j * IDX_COLS, IDX_COLS)],
                sem,
            )

    def wait_gathers(idx_v, rows_v, sem):
        for j in range(K_SUB):
            pltpu.make_async_copy(
                table_hbm.at[idx_v.at[j]],
                rows_v.at[pl.ds(j * IDX_COLS, IDX_COLS)],
                sem,
            ).wait()

    def fire_scatter(i, rows_v, sem):
        return pltpu.async_copy(
            rows_v, out_hbm.at[pl.ds(row_base + i * CHUNK, CHUNK)], sem)

    def wait_scatter(i, rows_v, sem):
        pltpu.make_async_copy(
            rows_v, out_hbm.at[pl.ds(row_base + i * CHUNK, CHUNK)], sem,
        ).wait()

    load_idx(0, idx0)
    fire_gathers(idx0, rows0, sem_g0)

    def body(k, carry):
        ia = 2 * k
        ib = 2 * k + 1
        wait_gathers(idx0, rows0, sem_g0)
        fire_scatter(ia, rows0, sem_s0)

        @pl.when(k > 0)
        def _():
            wait_scatter(ib, rows1, sem_s1)
        load_idx(ib, idx1)
        fire_gathers(idx1, rows1, sem_g1)

        wait_gathers(idx1, rows1, sem_g1)
        fire_scatter(ib, rows1, sem_s1)

        wait_scatter(ia, rows0, sem_s0)

        @pl.when(k < N_CHUNK_PAIRS - 1)
        def _():
            load_idx(ia + 2, idx0)
            fire_gathers(idx0, rows0, sem_g0)

        return carry

    lax.fori_loop(0, N_CHUNK_PAIRS, body, 0)
    wait_scatter(N_CHUNKS - 1, rows1, sem_s1)


def kernel(binS, binU, binF, embS, embU, embF):
    table = _build_table(embS, embU, embF)
    cidx = _combine_idx(binS, binU, binF)
    out = _sc_lookup(table, cidx)
    return out.reshape(BATCH, HIST, D_MODEL)
